# baseline (speedup 1.0000x reference)
import functools

import jax
import jax.numpy as jnp
from jax import lax
from jax.experimental import pallas as pl
from jax.experimental.pallas import tpu as pltpu
from jax.experimental.pallas import tpu_sc as plsc

B = 512
D = 512
C = 3
DELTA = 1.5

_SC_MESH = plsc.VectorSubcoreMesh(core_axis_name="c", subcore_axis_name="s")


@functools.partial(
    pl.kernel,
    mesh=_SC_MESH,
    compiler_params=pltpu.CompilerParams(use_tc_tiling_on_sc=False,
                                         needs_layout_passes=False),
    out_type=jax.ShapeDtypeStruct((3 * C * B,), jnp.float32),
    scratch_types=[
        pltpu.VMEM((B * C,), jnp.float32),
        pltpu.VMEM((B,), jnp.float32),
    ],
)
def _label_prep_sc(tl_hbm, al_hbm, vl_hbm, out_hbm, flat_v, row_v):
    wid = lax.axis_index("s") * 2 + lax.axis_index("c")
    lanes = lax.iota(jnp.int32, 16)

    for m, l_hbm in enumerate((tl_hbm, al_hbm, vl_hbm)):
        for k in range(C):
            @pl.when(wid == C * m + k)
            def _(l_hbm=l_hbm, m=m, k=k):
                pltpu.sync_copy(l_hbm, flat_v)
                for cidx in range(B // 16):
                    idx = 48 * cidx + C * lanes + k
                    vals = plsc.load_gather(flat_v, [idx])
                    row_v[pl.ds(16 * cidx, 16)] = vals
                pltpu.sync_copy(row_v, out_hbm.at[pl.ds((C * m + k) * B, B)])


def _row_argmax_onehot(lT):
    l0, l1, l2 = lT[0], lT[1], lT[2]
    is0 = jnp.logical_and(l0 >= l1, l0 >= l2)
    is1 = jnp.logical_and(jnp.logical_not(is0), l1 >= l2)
    pred = jnp.where(is0, 0, jnp.where(is1, 1, 2))
    kiota = jax.lax.broadcasted_iota(jnp.int32, (C, lT.shape[1]), 0)
    return (kiota == pred[None, :]).astype(jnp.float32)


def _fused_kernel(tf_ref, af_ref, vf_ref, labsT_ref, fr_ref, lc_ref,
                  adj_ref, nf_ref):
    s = 0.5 / (3.0 * DELTA)

    wacc = jnp.zeros((B, B), dtype=jnp.float32)
    centers_sum = jnp.zeros((C, D), dtype=jnp.float32)

    for m, f_ref in enumerate((tf_ref, af_ref, vf_ref)):
        feats = f_ref[:]
        labT = labsT_ref[C * m:C * m + C, :]

        onehotT = _row_argmax_onehot(labT)

        logpT = jnp.log(labT)
        hT = jnp.sum(labT * logpT, axis=0, keepdims=True)
        ones = jnp.ones((1, B), dtype=jnp.float32)
        UT = jnp.concatenate([labT, logpT, hT, ones], axis=0)
        VT = jnp.concatenate([s * logpT, s * labT, -s * ones,
                              1.0 / 3.0 - s * hT], axis=0)
        T = jax.lax.dot_general(UT, VT, (((0,), (0,)), ((), ())),
                                preferred_element_type=jnp.float32)

        counts = jnp.sum(onehotT, axis=1)
        centers = jnp.dot(onehotT, feats,
                          preferred_element_type=jnp.float32)
        centers = centers / jnp.maximum(counts, 1.0)[:, None]
        centers_sum = centers_sum + centers

        f2k = jnp.sum(feats * feats, axis=1, keepdims=True)
        c2k = jnp.sum(centers * centers, axis=1, keepdims=True)
        onesC = jnp.ones((C, 1), dtype=jnp.float32)
        onesB = jnp.ones((B, 1), dtype=jnp.float32)
        lhs2 = jnp.concatenate([-0.5 * c2k, onesC], axis=1)
        rhs2 = jnp.concatenate([onesB, -0.5 * f2k], axis=1)
        GT = jax.lax.dot_general(centers, feats, (((1,), (1,)), ((), ())),
                                 preferred_element_type=jnp.float32)
        GT = GT + jax.lax.dot_general(lhs2, rhs2, (((1,), (1,)), ((), ())),
                                      preferred_element_type=jnp.float32)
        gT = jnp.exp(GT)

        dij = jax.lax.dot_general(gT, onehotT, (((0,), (0,)), ((), ())),
                                  preferred_element_type=jnp.float32)
        dji = jax.lax.dot_general(onehotT, gT, (((0,), (0,)), ((), ())),
                                  preferred_element_type=jnp.float32)

        wacc = wacc + jnp.maximum(T, 0.0) * dij * dji

    ri = jax.lax.broadcasted_iota(jnp.int32, (B, B), 0)
    ci = jax.lax.broadcasted_iota(jnp.int32, (B, B), 1)
    w = jnp.where(ri == ci, 0.0, wacc)

    fused = fr_ref[:]
    lcent = lc_ref[:]
    logitsT = jax.lax.dot_general(lcent, fused, (((1,), (1,)), ((), ())),
                                  preferred_element_type=jnp.float32)
    onehot_fT = _row_argmax_onehot(logitsT)

    avg_c = centers_sum * (1.0 / 3.0)
    fu2 = jnp.sum(fused * fused, axis=1)
    a2k = jnp.sum(avg_c * avg_c, axis=1, keepdims=True)
    onesBf = jnp.ones((B, 1), dtype=jnp.float32)
    GfT = jax.lax.dot_general(avg_c, fused, (((1,), (1,)), ((), ())),
                              preferred_element_type=jnp.float32)
    A2T = jax.lax.dot_general(a2k, onesBf, (((1,), (1,)), ((), ())),
                              preferred_element_type=jnp.float32)
    d2f = fu2 + jnp.sum(onehot_fT * (A2T - 2.0 * GfT), axis=0)
    wf = jnp.exp(-0.5 * d2f)

    RT = wf[None, :] * onehot_fT

    adj_ref[0:B, 0:B] = w
    adj_ref[0:B, B:B + C] = RT.T
    adj_ref[B:B + C, 0:B] = RT
    adj_ref[B:B + C, B:B + C] = jnp.zeros((C, C), dtype=jnp.float32)

    nf_ref[0:B, :] = fused
    nf_ref[B:B + C, :] = lcent


@functools.partial(jax.jit)
def kernel(text_features, audio_features, vision_features, text_labels,
           audio_labels, vision_labels, fused_representations,
           learnable_class_centers):
    n = B + C
    labsT = _label_prep_sc(text_labels.reshape(B * C),
                           audio_labels.reshape(B * C),
                           vision_labels.reshape(B * C)).reshape(3 * C, B)
    adj, node_features = pl.pallas_call(
        _fused_kernel,
        out_shape=(
            jax.ShapeDtypeStruct((n, n), jnp.float32),
            jax.ShapeDtypeStruct((n, D), jnp.float32),
        ),
    )(text_features, audio_features, vision_features, labsT,
      fused_representations, learnable_class_centers)
    return adj, node_features


# speedup vs baseline: 3.6574x; 3.6574x over previous
import functools

import jax
import jax.numpy as jnp
from jax.experimental import pallas as pl
from jax.experimental.pallas import tpu as pltpu

B = 512
D = 512
C = 3
DELTA = 1.5


def _row_argmax_onehot(lT):
    l0, l1, l2 = lT[0], lT[1], lT[2]
    is0 = jnp.logical_and(l0 >= l1, l0 >= l2)
    is1 = jnp.logical_and(jnp.logical_not(is0), l1 >= l2)
    pred = jnp.where(is0, 0, jnp.where(is1, 1, 2))
    kiota = jax.lax.broadcasted_iota(jnp.int32, (C, lT.shape[1]), 0)
    return (kiota == pred[None, :]).astype(jnp.float32)


def _fused_kernel(tf_ref, af_ref, vf_ref, labsT_ref, fr_ref, lc_ref,
                  adj_ref, nf_ref):
    s = 0.5 / (3.0 * DELTA)

    onesC = jnp.ones((C, 1), dtype=jnp.float32)
    onesB = jnp.ones((B, 1), dtype=jnp.float32)
    ones = jnp.ones((1, B), dtype=jnp.float32)
    phase_a = []
    for m in range(3):
        labT = labsT_ref[C * m:C * m + C, :]
        onehotT = _row_argmax_onehot(labT)
        logpT = jnp.log(labT)
        hT = jnp.sum(labT * logpT, axis=0, keepdims=True)
        UT = jnp.concatenate([labT, logpT, hT, ones], axis=0)
        VT = jnp.concatenate([s * logpT, s * labT, -s * ones,
                              1.0 / 3.0 - s * hT], axis=0)
        T = jax.lax.dot_general(UT, VT, (((0,), (0,)), ((), ())),
                                preferred_element_type=jnp.float32)
        phase_a.append((T, onehotT))

    centers_sum = jnp.zeros((C, D), dtype=jnp.float32)
    phase_b = []
    for m, f_ref in enumerate((tf_ref, af_ref, vf_ref)):
        feats = f_ref[:]
        T, onehotT = phase_a[m]
        counts = jnp.sum(onehotT, axis=1)
        centers = jnp.dot(onehotT, feats,
                          preferred_element_type=jnp.float32)
        centers = centers / jnp.maximum(counts, 1.0)[:, None]
        centers_sum = centers_sum + centers
        f2k = jnp.sum(feats * feats, axis=1, keepdims=True)
        c2k = jnp.sum(centers * centers, axis=1, keepdims=True)
        lhs2 = jnp.concatenate([-0.5 * c2k, onesC], axis=1)
        rhs2 = jnp.concatenate([onesB, -0.5 * f2k], axis=1)
        GT = jax.lax.dot_general(centers, feats, (((1,), (1,)), ((), ())),
                                 preferred_element_type=jnp.float32)
        GT = GT + jax.lax.dot_general(lhs2, rhs2, (((1,), (1,)), ((), ())),
                                      preferred_element_type=jnp.float32)
        phase_b.append(jnp.exp(GT))

    wacc = jnp.zeros((B, B), dtype=jnp.float32)
    for m in range(3):
        T, onehotT = phase_a[m]
        gT = phase_b[m]
        dij = jax.lax.dot_general(gT, onehotT, (((0,), (0,)), ((), ())),
                                  preferred_element_type=jnp.float32)
        dji = jax.lax.dot_general(onehotT, gT, (((0,), (0,)), ((), ())),
                                  preferred_element_type=jnp.float32)
        wacc = wacc + jnp.maximum(T, 0.0) * dij * dji

    ri = jax.lax.broadcasted_iota(jnp.int32, (B, B), 0)
    ci = jax.lax.broadcasted_iota(jnp.int32, (B, B), 1)
    w = jnp.where(ri == ci, 0.0, wacc)

    fused = fr_ref[:]
    lcent = lc_ref[:]
    logitsT = jax.lax.dot_general(lcent, fused, (((1,), (1,)), ((), ())),
                                  preferred_element_type=jnp.float32)
    onehot_fT = _row_argmax_onehot(logitsT)

    avg_c = centers_sum * (1.0 / 3.0)
    fu2 = jnp.sum(fused * fused, axis=1)
    a2k = jnp.sum(avg_c * avg_c, axis=1, keepdims=True)
    onesBf = jnp.ones((B, 1), dtype=jnp.float32)
    GfT = jax.lax.dot_general(avg_c, fused, (((1,), (1,)), ((), ())),
                              preferred_element_type=jnp.float32)
    A2T = jax.lax.dot_general(a2k, onesBf, (((1,), (1,)), ((), ())),
                              preferred_element_type=jnp.float32)
    d2f = fu2 + jnp.sum(onehot_fT * (A2T - 2.0 * GfT), axis=0)
    wf = jnp.exp(-0.5 * d2f)

    RT = wf[None, :] * onehot_fT

    adj_ref[0:B, 0:B] = w
    adj_ref[0:B, B:B + C] = RT.T
    adj_ref[B:B + C, 0:B] = RT
    adj_ref[B:B + C, B:B + C] = jnp.zeros((C, C), dtype=jnp.float32)

    nf_ref[0:B, :] = fused
    nf_ref[B:B + C, :] = lcent


@functools.partial(jax.jit)
def kernel(text_features, audio_features, vision_features, text_labels,
           audio_labels, vision_labels, fused_representations,
           learnable_class_centers):
    n = B + C
    labsT = jnp.concatenate(
        [text_labels.T, audio_labels.T, vision_labels.T], axis=0)
    adj, node_features = pl.pallas_call(
        _fused_kernel,
        out_shape=(
            jax.ShapeDtypeStruct((n, n), jnp.float32),
            jax.ShapeDtypeStruct((n, D), jnp.float32),
        ),
    )(text_features, audio_features, vision_features, labsT,
      fused_representations, learnable_class_centers)
    return adj, node_features


# speedup vs baseline: 4.3277x; 1.1833x over previous
import functools

import jax
import jax.numpy as jnp
from jax.experimental import pallas as pl
from jax.experimental.pallas import tpu as pltpu

B = 512
D = 512
C = 3
DELTA = 1.5


def _row_argmax_onehot(lT):
    l0, l1, l2 = lT[0], lT[1], lT[2]
    is0 = jnp.logical_and(l0 >= l1, l0 >= l2)
    is1 = jnp.logical_and(jnp.logical_not(is0), l1 >= l2)
    pred = jnp.where(is0, 0, jnp.where(is1, 1, 2))
    kiota = jax.lax.broadcasted_iota(jnp.int32, (C, lT.shape[1]), 0)
    return (kiota == pred[None, :]).astype(jnp.float32)


def _fused_kernel(tf_ref, af_ref, vf_ref, labsT_ref, fr_ref, lc_ref,
                  adj_ref, nf_ref):
    s = 0.5 / (3.0 * DELTA)

    onesC = jnp.ones((C, 1), dtype=jnp.float32)
    onesB = jnp.ones((B, 1), dtype=jnp.float32)
    ones = jnp.ones((1, B), dtype=jnp.float32)
    phase_a = []
    for m in range(3):
        labT = labsT_ref[C * m:C * m + C, :]
        onehotT = _row_argmax_onehot(labT)
        logpT = jnp.log(labT)
        hT = jnp.sum(labT * logpT, axis=0, keepdims=True)
        UT = jnp.concatenate([labT, logpT, hT, ones], axis=0)
        VT = jnp.concatenate([s * logpT, s * labT, -s * ones,
                              1.0 / 3.0 - s * hT], axis=0)
        T = jax.lax.dot_general(UT, VT, (((0,), (0,)), ((), ())),
                                preferred_element_type=jnp.float32)
        phase_a.append((T, onehotT))

    centers_sum = jnp.zeros((C, D), dtype=jnp.float32)
    phase_b = []
    for m, f_ref in enumerate((tf_ref, af_ref, vf_ref)):
        feats = f_ref[:]
        T, onehotT = phase_a[m]
        counts = jnp.sum(onehotT, axis=1)
        centers = jnp.dot(onehotT, feats,
                          preferred_element_type=jnp.float32)
        centers = centers / jnp.maximum(counts, 1.0)[:, None]
        centers_sum = centers_sum + centers
        f2k = jnp.sum(feats * feats, axis=1, keepdims=True)
        c2k = jnp.sum(centers * centers, axis=1, keepdims=True)
        lhs2 = jnp.concatenate([-0.5 * c2k, onesC], axis=1)
        rhs2 = jnp.concatenate([onesB, -0.5 * f2k], axis=1)
        GT = jax.lax.dot_general(centers, feats, (((1,), (1,)), ((), ())),
                                 preferred_element_type=jnp.float32)
        GT = GT + jax.lax.dot_general(lhs2, rhs2, (((1,), (1,)), ((), ())),
                                      preferred_element_type=jnp.float32)
        phase_b.append(jnp.exp(GT))

    wacc = jnp.zeros((B, B), dtype=jnp.float32)
    for m in range(3):
        T, onehotT = phase_a[m]
        gT = phase_b[m]
        dij = jax.lax.dot_general(gT, onehotT, (((0,), (0,)), ((), ())),
                                  preferred_element_type=jnp.float32)
        dji = jax.lax.dot_general(onehotT, gT, (((0,), (0,)), ((), ())),
                                  preferred_element_type=jnp.float32)
        wacc = wacc + jnp.maximum(T, 0.0) * dij * dji

    ri = jax.lax.broadcasted_iota(jnp.int32, (B, B), 0)
    ci = jax.lax.broadcasted_iota(jnp.int32, (B, B), 1)
    w = jnp.where(ri == ci, 0.0, wacc)

    fused = fr_ref[:]
    lcent = lc_ref[:]
    logitsT = jax.lax.dot_general(lcent, fused, (((1,), (1,)), ((), ())),
                                  preferred_element_type=jnp.float32)
    onehot_fT = _row_argmax_onehot(logitsT)

    avg_c = centers_sum * (1.0 / 3.0)
    fu2 = jnp.sum(fused * fused, axis=1)
    a2k = jnp.sum(avg_c * avg_c, axis=1, keepdims=True)
    onesBf = jnp.ones((B, 1), dtype=jnp.float32)
    GfT = jax.lax.dot_general(avg_c, fused, (((1,), (1,)), ((), ())),
                              preferred_element_type=jnp.float32)
    A2T = jax.lax.dot_general(a2k, onesBf, (((1,), (1,)), ((), ())),
                              preferred_element_type=jnp.float32)
    d2f = fu2 + jnp.sum(onehot_fT * (A2T - 2.0 * GfT), axis=0)
    wf = jnp.exp(-0.5 * d2f)

    RT = wf[None, :] * onehot_fT

    adj_ref[0:B, 0:B] = w
    adj_ref[0:B, B:B + C] = RT.T
    adj_ref[B:B + C, 0:B] = RT
    adj_ref[B:B + C, B:B + C] = jnp.zeros((C, C), dtype=jnp.float32)

    nf_ref[0:B, :] = fused
    nf_ref[B:B + C, :] = lcent


@functools.partial(jax.jit)
def kernel(text_features, audio_features, vision_features, text_labels,
           audio_labels, vision_labels, fused_representations,
           learnable_class_centers):
    n = B + C
    labsT = jnp.concatenate(
        [text_labels.T, audio_labels.T, vision_labels.T], axis=0)
    adj, node_features = pl.pallas_call(
        _fused_kernel,
        compiler_params=pltpu.CompilerParams(
            allow_input_fusion=(False, False, False, True, False, False)),
        out_shape=(
            jax.ShapeDtypeStruct((n, n), jnp.float32),
            jax.ShapeDtypeStruct((n, D), jnp.float32),
        ),
    )(text_features, audio_features, vision_features, labsT,
      fused_representations, learnable_class_centers)
    return adj, node_features
